# baseline (speedup 1.0000x reference)
import functools

import jax
import jax.numpy as jnp
from jax import lax
from jax.experimental import pallas as pl
from jax.experimental.pallas import tpu as pltpu
from jax.experimental.pallas import tpu_sc as plsc

IN_CH = 4096
N_ROWS = 8192
L = 16
NC, NS = 2, 16
NW = NC * NS
ROWS_PER_W = N_ROWS // NW
C = 8
CHUNKS = ROWS_PER_W // C
NPAIR = CHUNKS // 2
NQ = 4
HW = IN_CH // NQ
HGROUPS = HW // L

_mesh = plsc.VectorSubcoreMesh(
    core_axis_name="c", subcore_axis_name="s", num_cores=NC, num_subcores=NS)


@functools.partial(
    pl.kernel,
    out_type=jax.ShapeDtypeStruct((N_ROWS, IN_CH), jnp.float32),
    mesh=_mesh,
    compiler_params=pltpu.CompilerParams(needs_layout_passes=False),
    scratch_types=[
        pltpu.VMEM((IN_CH,), jnp.int32),
        pltpu.VMEM((C, IN_CH), jnp.float32),
        pltpu.VMEM((C, IN_CH), jnp.float32),
        pltpu.VMEM((C, HW), jnp.float32),
        pltpu.VMEM((C, HW), jnp.float32),
        pltpu.VMEM((C, HW), jnp.float32),
        pltpu.VMEM((C, HW), jnp.float32),
        pltpu.SemaphoreType.DMA,
        pltpu.SemaphoreType.DMA,
        pltpu.SemaphoreType.DMA,
        pltpu.SemaphoreType.DMA,
        pltpu.SemaphoreType.DMA,
        pltpu.SemaphoreType.DMA,
    ],
)
def _permute(x_hbm, p_hbm, out_hbm, p_v, in0, in1, oh0, oh1, oh2, oh3,
             sem_in0, sem_in1, sem_out0, sem_out1, sem_out2, sem_out3):
    wid = lax.axis_index("s") * NC + lax.axis_index("c")
    row0 = wid * ROWS_PER_W
    pltpu.sync_copy(p_hbm, p_v)

    ohs = (oh0, oh1, oh2, oh3)
    sem_outs = (sem_out0, sem_out1, sem_out2, sem_out3)

    def in_copy(ci, buf, sem):
        r0 = pl.multiple_of(row0 + ci * C, C)
        return pltpu.make_async_copy(x_hbm.at[pl.ds(r0, C), :], buf, sem)

    def out_copy(ci, h, buf, sem):
        r0 = pl.multiple_of(row0 + ci * C, C)
        return pltpu.make_async_copy(
            buf, out_hbm.at[pl.ds(r0, C), pl.ds(h * HW, HW)], sem)

    def permute_half(src, dst, h):
        @plsc.parallel_loop(0, HGROUPS, unroll=4)
        def _(g):
            col = pl.multiple_of(h * HW + g * L, L)
            idx = p_v[pl.ds(col, L)]
            for r in range(C):
                rvec = jnp.full((L,), r, jnp.int32)
                val = plsc.load_gather(src, [rvec, idx])
                dst[r, pl.ds(g * L, L)] = val

    def do_chunk(ci, ibuf, isem):
        in_copy(ci, ibuf, isem).wait()
        nbuf, nsem = (in1, sem_in1) if ibuf is in0 else (in0, sem_in0)

        @pl.when(ci + 1 < CHUNKS)
        def _():
            in_copy(ci + 1, nbuf, nsem).start()

        for h in range(NQ):
            @pl.when(ci > 0)
            def _():
                out_copy(ci - 1, h, ohs[h], sem_outs[h]).wait()

            permute_half(ibuf, ohs[h], h)
            out_copy(ci, h, ohs[h], sem_outs[h]).start()

    in_copy(0, in0, sem_in0).start()

    def pair_body(i, carry):
        do_chunk(i * 2, in0, sem_in0)
        do_chunk(i * 2 + 1, in1, sem_in1)
        return carry

    lax.fori_loop(0, NPAIR, pair_body, 0)

    for h in range(NQ):
        out_copy(CHUNKS - 1, h, ohs[h], sem_outs[h]).wait()


def kernel(x, p):
    out = _permute(x, p.astype(jnp.int32))
    return (out, 0)


# speedup vs baseline: 1.0114x; 1.0114x over previous
import functools

import jax
import jax.numpy as jnp
from jax import lax
from jax.experimental import pallas as pl
from jax.experimental.pallas import tpu as pltpu
from jax.experimental.pallas import tpu_sc as plsc

IN_CH = 4096
N_ROWS = 8192
L = 16
NC, NS = 2, 16
NW = NC * NS
ROWS_PER_W = N_ROWS // NW
C = 8
CHUNKS = ROWS_PER_W // C
NPAIR = CHUNKS // 2
NQ = 2
HW = IN_CH // NQ
HGROUPS = HW // L

_mesh = plsc.VectorSubcoreMesh(
    core_axis_name="c", subcore_axis_name="s", num_cores=NC, num_subcores=NS)


@functools.partial(
    pl.kernel,
    out_type=jax.ShapeDtypeStruct((N_ROWS, IN_CH), jnp.float32),
    mesh=_mesh,
    compiler_params=pltpu.CompilerParams(needs_layout_passes=False),
    scratch_types=[
        pltpu.VMEM((IN_CH,), jnp.int32),
        pltpu.VMEM((C, IN_CH), jnp.float32),
        pltpu.VMEM((C, IN_CH), jnp.float32),
        pltpu.VMEM((C, HW), jnp.float32),
        pltpu.VMEM((C, HW), jnp.float32),
        pltpu.SemaphoreType.DMA,
        pltpu.SemaphoreType.DMA,
        pltpu.SemaphoreType.DMA,
        pltpu.SemaphoreType.DMA,
    ],
)
def _permute(x_hbm, p_hbm, out_hbm, p_v, in0, in1, oh0, oh1,
             sem_in0, sem_in1, sem_out0, sem_out1):
    wid = lax.axis_index("s") * NC + lax.axis_index("c")
    row0 = wid * ROWS_PER_W
    pltpu.sync_copy(p_hbm, p_v)

    ohs = (oh0, oh1)
    sem_outs = (sem_out0, sem_out1)

    def in_copy(ci, buf, sem):
        r0 = pl.multiple_of(row0 + ci * C, C)
        return pltpu.make_async_copy(x_hbm.at[pl.ds(r0, C), :], buf, sem)

    def out_copy(ci, h, buf, sem):
        r0 = pl.multiple_of(row0 + ci * C, C)
        return pltpu.make_async_copy(
            buf, out_hbm.at[pl.ds(r0, C), pl.ds(h * HW, HW)], sem)

    def permute_half(src, dst, h):
        @plsc.parallel_loop(0, HGROUPS, unroll=4)
        def _(g):
            col = pl.multiple_of(h * HW + g * L, L)
            idx = p_v[pl.ds(col, L)]
            for r in range(C):
                rvec = jnp.full((L,), r, jnp.int32)
                val = plsc.load_gather(src, [rvec, idx])
                dst[r, pl.ds(g * L, L)] = val

    def do_chunk(ci, ibuf, isem):
        in_copy(ci, ibuf, isem).wait()
        nbuf, nsem = (in1, sem_in1) if ibuf is in0 else (in0, sem_in0)

        @pl.when(ci + 1 < CHUNKS)
        def _():
            in_copy(ci + 1, nbuf, nsem).start()

        for h in range(NQ):
            @pl.when(ci > 0)
            def _():
                out_copy(ci - 1, h, ohs[h], sem_outs[h]).wait()

            permute_half(ibuf, ohs[h], h)
            out_copy(ci, h, ohs[h], sem_outs[h]).start()

    in_copy(0, in0, sem_in0).start()

    def pair_body(i, carry):
        do_chunk(i * 2, in0, sem_in0)
        do_chunk(i * 2 + 1, in1, sem_in1)
        return carry

    lax.fori_loop(0, NPAIR, pair_body, 0)

    for h in range(NQ):
        out_copy(CHUNKS - 1, h, ohs[h], sem_outs[h]).wait()


def kernel(x, p):
    out = _permute(x, p.astype(jnp.int32))
    return (out, 0)


# speedup vs baseline: 1.0225x; 1.0110x over previous
import functools

import jax
import jax.numpy as jnp
from jax import lax
from jax.experimental import pallas as pl
from jax.experimental.pallas import tpu as pltpu
from jax.experimental.pallas import tpu_sc as plsc

IN_CH = 4096
N_ROWS = 8192
L = 16
NC, NS = 2, 16
NW = NC * NS
ROWS_PER_W = N_ROWS // NW
C = 8
CHUNKS = ROWS_PER_W // C
NPAIR = CHUNKS // 2
NQ = 2
HW = IN_CH // NQ
HGROUPS = HW // L

_mesh = plsc.VectorSubcoreMesh(
    core_axis_name="c", subcore_axis_name="s", num_cores=NC, num_subcores=NS)


@functools.partial(
    pl.kernel,
    out_type=jax.ShapeDtypeStruct((N_ROWS, IN_CH), jnp.float32),
    mesh=_mesh,
    compiler_params=pltpu.CompilerParams(needs_layout_passes=False),
    scratch_types=[
        pltpu.VMEM((IN_CH,), jnp.int32),
        pltpu.VMEM((C, IN_CH), jnp.float32),
        pltpu.VMEM((C, IN_CH), jnp.float32),
        pltpu.VMEM((C, HW), jnp.float32),
        pltpu.VMEM((C, HW), jnp.float32),
        pltpu.SemaphoreType.DMA,
        pltpu.SemaphoreType.DMA,
        pltpu.SemaphoreType.DMA,
        pltpu.SemaphoreType.DMA,
        pltpu.SemaphoreType.DMA,
        pltpu.SemaphoreType.DMA,
    ],
)
def _permute(x_hbm, p_hbm, out_hbm, p_v, in0, in1, oh0, oh1,
             sem_in0, sem_in1, sem_in0b, sem_in1b, sem_out0, sem_out1):
    wid = lax.axis_index("s") * NC + lax.axis_index("c")
    row0 = wid * ROWS_PER_W
    pltpu.sync_copy(p_hbm, p_v)

    ohs = (oh0, oh1)
    sem_outs = (sem_out0, sem_out1)

    def in_copies(ci, buf, sem, semb):
        r0 = pl.multiple_of(row0 + ci * C, C)
        return (
            pltpu.make_async_copy(
                x_hbm.at[pl.ds(r0, C), pl.ds(0, HW)],
                buf.at[:, pl.ds(0, HW)], sem),
            pltpu.make_async_copy(
                x_hbm.at[pl.ds(r0, C), pl.ds(HW, HW)],
                buf.at[:, pl.ds(HW, HW)], semb),
        )

    def out_copy(ci, h, buf, sem):
        r0 = pl.multiple_of(row0 + ci * C, C)
        return pltpu.make_async_copy(
            buf, out_hbm.at[pl.ds(r0, C), pl.ds(h * HW, HW)], sem)

    def permute_half(src, dst, h):
        @plsc.parallel_loop(0, HGROUPS, unroll=4)
        def _(g):
            col = pl.multiple_of(h * HW + g * L, L)
            idx = p_v[pl.ds(col, L)]
            for r in range(C):
                rvec = jnp.full((L,), r, jnp.int32)
                val = plsc.load_gather(src, [rvec, idx])
                dst[r, pl.ds(g * L, L)] = val

    def do_chunk(ci, ibuf, isem, isemb):
        for cp in in_copies(ci, ibuf, isem, isemb):
            cp.wait()
        if ibuf is in0:
            nbuf, nsem, nsemb = in1, sem_in1, sem_in1b
        else:
            nbuf, nsem, nsemb = in0, sem_in0, sem_in0b

        @pl.when(ci + 1 < CHUNKS)
        def _():
            for cp in in_copies(ci + 1, nbuf, nsem, nsemb):
                cp.start()

        for h in range(NQ):
            @pl.when(ci > 0)
            def _():
                out_copy(ci - 1, h, ohs[h], sem_outs[h]).wait()

            permute_half(ibuf, ohs[h], h)
            out_copy(ci, h, ohs[h], sem_outs[h]).start()

    for cp in in_copies(0, in0, sem_in0, sem_in0b):
        cp.start()

    def pair_body(i, carry):
        do_chunk(i * 2, in0, sem_in0, sem_in0b)
        do_chunk(i * 2 + 1, in1, sem_in1, sem_in1b)
        return carry

    lax.fori_loop(0, NPAIR, pair_body, 0)

    for h in range(NQ):
        out_copy(CHUNKS - 1, h, ohs[h], sem_outs[h]).wait()


def kernel(x, p):
    out = _permute(x, p.astype(jnp.int32))
    return (out, 0)
